# baseline (speedup 1.0000x reference)
import functools

import jax
import jax.numpy as jnp
from jax import lax
from jax.experimental import pallas as pl
from jax.experimental.pallas import tpu as pltpu
from jax.experimental.pallas import tpu_sc as plsc

N_OLD = 900000
DIM = 64
N = 16384 * 50
NC = 2
NS = 16
NW = NC * NS
C = N // NW
NB = 128
NBLK = C // NB
LIST_PAD = 160
NT = N // NB
KC = DIM // 8

_mesh = plsc.VectorSubcoreMesh(core_axis_name="c", subcore_axis_name="s")
_params = pltpu.CompilerParams(needs_layout_passes=False,
                               use_tc_tiling_on_sc=False)


def _wid():
    return lax.axis_index("s") * NC + lax.axis_index("c")


@functools.partial(
    pl.kernel,
    out_type=(
        jax.ShapeDtypeStruct((N + C + 32,), jnp.int32),
        jax.ShapeDtypeStruct((N + C + 32,), jnp.int32),
        jax.ShapeDtypeStruct((NW * 16,), jnp.int32),
    ),
    mesh=_mesh,
    compiler_params=_params,
    scratch_types=[
        pltpu.VMEM((C,), jnp.int32),
        pltpu.VMEM((C + LIST_PAD,), jnp.int32),
        pltpu.VMEM((C + LIST_PAD,), jnp.int32),
        pltpu.VMEM((16,), jnp.int32),
    ],
)
def _prep_kernel(x_hbm, old_hbm, new_hbm, counts_hbm,
                 chunk_v, old_v, new_v, cnt_v):
    w = _wid()
    pltpu.sync_copy(x_hbm.at[pl.ds(w * C, C)], chunk_v)

    thr = jnp.full((16,), N_OLD, jnp.int32)

    def part_body(i, carry):
        o, nf = carry
        v = chunk_v[pl.ds(i * 16, 16)]
        m = v < thr
        plsc.store_compressed(old_v.at[pl.ds(o, 16)], v, mask=m)
        plsc.store_compressed(new_v.at[pl.ds(nf, 16)], v - thr,
                              mask=jnp.logical_not(m))
        c = jnp.sum(lax.shift_right_logical(v - thr, 31))
        return o + c, nf + (16 - c)

    n_old, _ = lax.fori_loop(0, C // 16, part_body,
                             (jnp.int32(0), jnp.int32(0)))

    pltpu.sync_copy(old_v.at[pl.ds(0, C)], old_hbm.at[pl.ds(w * C, C)])
    pltpu.sync_copy(new_v.at[pl.ds(0, C)], new_hbm.at[pl.ds(w * C, C)])
    cnt_v[...] = jnp.full((16,), 0, jnp.int32) + n_old
    pltpu.sync_copy(cnt_v, counts_hbm.at[pl.ds(w * 16, 16)])


@functools.partial(
    pl.kernel,
    out_type=jax.ShapeDtypeStruct((N * DIM,), jnp.float32),
    mesh=_mesh,
    compiler_params=_params,
    scratch_types=[
        pltpu.VMEM((C + 32,), jnp.int32),
        pltpu.VMEM((C + 32,), jnp.int32),
        pltpu.VMEM((NW * 16,), jnp.int32),
        pltpu.VMEM((NB, DIM), jnp.float32),
        pltpu.VMEM((NB, DIM), jnp.float32),
        pltpu.VMEM((NB,), jnp.int32),
        pltpu.VMEM((KC * 8 * NB,), jnp.float32),
        pltpu.SemaphoreType.DMA,
        pltpu.SemaphoreType.DMA,
    ],
)
def _main_kernel(wold_hbm, wnew_hbm, old_hbm, new_hbm, counts_hbm, out_hbm,
                 gl, piece_v, counts_v, rows_a, rows_b, sidx, tile_v,
                 sem_a, sem_b):
    w = _wid()
    iota16 = lax.iota(jnp.int32, 16)
    pltpu.sync_copy(counts_hbm, counts_v)

    k_total = jnp.int32(0)
    for j in range(NW):
        k_total = k_total + jnp.max(counts_v[pl.ds(j * 16, 16)])

    a = w * C
    m = jnp.clip(k_total - a, 0, C)

    def chunk_cnt(j):
        return jnp.max(counts_v[pl.ds(j * 16, 16)])

    def copy_piece(hbm, src, dst, ln):
        s8 = pl.multiple_of(jnp.bitwise_and(src, jnp.int32(-8)), 8)
        lead = src - s8
        pltpu.sync_copy(hbm.at[pl.ds(s8, C + 16)], piece_v.at[pl.ds(0, C + 16)])

        def cp(q, c2):
            u = plsc.load_gather(piece_v, [lead + 16 * q + iota16])
            gl[pl.ds(dst + 16 * q, 16)] = u
            return c2

        lax.fori_loop(0, (ln + 15) // 16, cp, jnp.int32(0))

    def piece_old(j, p_run):
        cj = chunk_cnt(j)
        lo = jnp.maximum(p_run, a)
        hi = jnp.minimum(p_run + cj, a + m)

        @pl.when(hi > lo)
        def _():
            copy_piece(old_hbm, j * C + (lo - p_run), lo - a, hi - lo)

        return p_run + cj

    lax.fori_loop(0, NW, piece_old, jnp.int32(0))

    a_n = a + m - k_total

    def piece_new(j, p_run):
        cj = C - chunk_cnt(j)
        lo = jnp.maximum(p_run, a_n)
        hi = jnp.minimum(p_run + cj, a_n + (C - m))

        @pl.when(hi > lo)
        def _():
            copy_piece(new_hbm, j * C + (lo - p_run), m + (lo - a_n), hi - lo)

        return p_run + cj

    lax.fori_loop(0, NW, piece_new, jnp.int32(0))

    def transpose_write(t, merge, m_s):
        def tk(k, c):
            colb = 8 * k

            def write_tile():
                tg = w * NBLK + t
                pltpu.sync_copy(
                    tile_v.at[pl.ds(0, 8 * NB)],
                    out_hbm.at[pl.ds((k * NT + tg) * (8 * NB), 8 * NB)])

            for cl in range(8):
                colv = jnp.full((16,), 0, jnp.int32) + (colb + cl)
                for g in range(8):
                    rowv = iota16 + (16 * g)
                    va = plsc.load_gather(rows_a, [rowv, colv])
                    if merge:
                        vb = plsc.load_gather(rows_b, [rowv, colv])
                        keep_old = (rowv + t * NB) < (jnp.full((16,), 0, jnp.int32) + m_s)
                        va = jnp.where(keep_old, va, vb)
                    tile_v[pl.ds(cl * NB + 16 * g, 16)] = va
            write_tile()
            return c

        lax.fori_loop(0, KC, tk, jnp.int32(0))

    def blk(i, table):
        bb = i * NB
        pltpu.async_copy(table.at[gl.at[pl.ds(bb, NB)]], rows_a, sem_a).wait()
        transpose_write(i, False, jnp.int32(0))

    split_lo = lax.shift_right_logical(m, 7)
    split_hi = lax.shift_right_logical(m + 127, 7)

    def blk_old(i, c):
        blk(i, wold_hbm)
        return c

    def blk_new(i, c):
        blk(i, wnew_hbm)
        return c

    lax.fori_loop(0, split_lo, blk_old, jnp.int32(0))
    lax.fori_loop(split_hi, NBLK, blk_new, jnp.int32(0))

    @pl.when(jnp.bitwise_and(m, 127) != 0)
    def _():
        sb = split_lo
        bb = sb * NB
        pltpu.async_copy(wold_hbm.at[gl.at[pl.ds(bb, NB)]], rows_a,
                         sem_a).wait()
        for g in range(8):
            v = gl[pl.ds(bb + 16 * g, 16)]
            pos = iota16 + (bb + 16 * g)
            is_new = pos >= (jnp.full((16,), 0, jnp.int32) + m)
            sidx[pl.ds(16 * g, 16)] = jnp.where(is_new, v, 0)
        pltpu.async_copy(wnew_hbm.at[sidx], rows_b, sem_b).wait()
        transpose_write(sb, True, m)


def kernel(x, W_old, W_new):
    flat = x.reshape(-1).astype(jnp.int32)
    old_p, new_p, counts = _prep_kernel(flat)
    out1 = _main_kernel(W_old, W_new, old_p, new_p, counts)
    return (out1.reshape(KC, NT, 8, NB)
            .transpose(1, 3, 0, 2)
            .reshape(N, DIM))


# speedup vs baseline: 1.1677x; 1.1677x over previous
import functools

import jax
import jax.numpy as jnp
from jax import lax
from jax.experimental import pallas as pl
from jax.experimental.pallas import tpu as pltpu
from jax.experimental.pallas import tpu_sc as plsc

N_OLD = 900000
DIM = 64
N = 16384 * 50
NC = 2
NS = 16
NW = NC * NS
C = N // NW
NB = 128
NBLK = C // NB
LIST_PAD = 160
NT = N // NB
KC = DIM // 8

_mesh = plsc.VectorSubcoreMesh(core_axis_name="c", subcore_axis_name="s")
_params = pltpu.CompilerParams(needs_layout_passes=False,
                               use_tc_tiling_on_sc=False)


def _wid():
    return lax.axis_index("s") * NC + lax.axis_index("c")


@functools.partial(
    pl.kernel,
    out_type=(
        jax.ShapeDtypeStruct((N + C + 32,), jnp.int32),
        jax.ShapeDtypeStruct((N + C + 32,), jnp.int32),
        jax.ShapeDtypeStruct((NW * 16,), jnp.int32),
    ),
    mesh=_mesh,
    compiler_params=_params,
    scratch_types=[
        pltpu.VMEM((C,), jnp.int32),
        pltpu.VMEM((C + LIST_PAD,), jnp.int32),
        pltpu.VMEM((C + LIST_PAD,), jnp.int32),
        pltpu.VMEM((16,), jnp.int32),
    ],
)
def _prep_kernel(x_hbm, old_hbm, new_hbm, counts_hbm,
                 chunk_v, old_v, new_v, cnt_v):
    w = _wid()
    pltpu.sync_copy(x_hbm.at[pl.ds(w * C, C)], chunk_v)

    thr = jnp.full((16,), N_OLD, jnp.int32)

    def part_body(i, carry):
        o, nf = carry
        v = chunk_v[pl.ds(i * 16, 16)]
        m = v < thr
        plsc.store_compressed(old_v.at[pl.ds(o, 16)], v, mask=m)
        plsc.store_compressed(new_v.at[pl.ds(nf, 16)], v - thr,
                              mask=jnp.logical_not(m))
        c = jnp.sum(lax.shift_right_logical(v - thr, 31))
        return o + c, nf + (16 - c)

    n_old, _ = lax.fori_loop(0, C // 16, part_body,
                             (jnp.int32(0), jnp.int32(0)))

    pltpu.sync_copy(old_v.at[pl.ds(0, C)], old_hbm.at[pl.ds(w * C, C)])
    pltpu.sync_copy(new_v.at[pl.ds(0, C)], new_hbm.at[pl.ds(w * C, C)])
    cnt_v[...] = jnp.full((16,), 0, jnp.int32) + n_old
    pltpu.sync_copy(cnt_v, counts_hbm.at[pl.ds(w * 16, 16)])


@functools.partial(
    pl.kernel,
    out_type=jax.ShapeDtypeStruct((N * DIM,), jnp.float32),
    mesh=_mesh,
    compiler_params=_params,
    scratch_types=[
        pltpu.VMEM((C + 32,), jnp.int32),
        pltpu.VMEM((C + 32,), jnp.int32),
        pltpu.VMEM((NW * 16,), jnp.int32),
        pltpu.VMEM((NB, DIM), jnp.float32),
        pltpu.VMEM((NB, DIM), jnp.float32),
        pltpu.VMEM((NB,), jnp.int32),
        pltpu.VMEM((KC * 8 * NB,), jnp.float32),
        pltpu.VMEM((KC * 8 * NB,), jnp.float32),
        pltpu.SemaphoreType.DMA,
        pltpu.SemaphoreType.DMA,
        pltpu.SemaphoreType.DMA,
        pltpu.SemaphoreType.DMA,
    ],
)
def _main_kernel(wold_hbm, wnew_hbm, old_hbm, new_hbm, counts_hbm, out_hbm,
                 gl, piece_v, counts_v, rows_a, rows_b, sidx, tiles_a,
                 tiles_b, sem_a, sem_b, sem_wa, sem_wb):
    w = _wid()
    iota16 = lax.iota(jnp.int32, 16)
    pltpu.sync_copy(counts_hbm, counts_v)

    k_total = jnp.int32(0)
    for j in range(NW):
        k_total = k_total + jnp.max(counts_v[pl.ds(j * 16, 16)])

    a = w * C
    m = jnp.clip(k_total - a, 0, C)

    def chunk_cnt(j):
        return jnp.max(counts_v[pl.ds(j * 16, 16)])

    def copy_piece(hbm, src, dst, ln):
        s8 = pl.multiple_of(jnp.bitwise_and(src, jnp.int32(-8)), 8)
        lead = src - s8
        pltpu.sync_copy(hbm.at[pl.ds(s8, C + 16)], piece_v.at[pl.ds(0, C + 16)])

        def cp(q, c2):
            u = plsc.load_gather(piece_v, [lead + 16 * q + iota16])
            gl[pl.ds(dst + 16 * q, 16)] = u
            return c2

        lax.fori_loop(0, (ln + 15) // 16, cp, jnp.int32(0))

    def piece_old(j, p_run):
        cj = chunk_cnt(j)
        lo = jnp.maximum(p_run, a)
        hi = jnp.minimum(p_run + cj, a + m)

        @pl.when(hi > lo)
        def _():
            copy_piece(old_hbm, j * C + (lo - p_run), lo - a, hi - lo)

        return p_run + cj

    lax.fori_loop(0, NW, piece_old, jnp.int32(0))

    a_n = a + m - k_total

    def piece_new(j, p_run):
        cj = C - chunk_cnt(j)
        lo = jnp.maximum(p_run, a_n)
        hi = jnp.minimum(p_run + cj, a_n + (C - m))

        @pl.when(hi > lo)
        def _():
            copy_piece(new_hbm, j * C + (lo - p_run), m + (lo - a_n), hi - lo)

        return p_run + cj

    lax.fori_loop(0, NW, piece_new, jnp.int32(0))

    def fill_tiles(t, rows_ref, tiles_ref, sem_w):
        def tk(k, c):
            colb = 8 * k
            for cl in range(8):
                colv = jnp.full((16,), 0, jnp.int32) + (colb + cl)
                for g in range(8):
                    rowv = iota16 + (16 * g)
                    va = plsc.load_gather(rows_ref, [rowv, colv])
                    tiles_ref[pl.ds(k * 1024 + cl * NB + 16 * g, 16)] = va
            tg = w * NBLK + t
            pltpu.async_copy(
                tiles_ref.at[pl.ds(k * 1024, 8 * NB)],
                out_hbm.at[pl.ds((k * NT + tg) * (8 * NB), 8 * NB)], sem_w)
            return c

        lax.fori_loop(0, KC, tk, jnp.int32(0))

    def drain_writes(tiles_ref, sem_w):
        def dk(k, c):
            pltpu.make_async_copy(tiles_ref.at[pl.ds(0, 8 * NB)],
                                  out_hbm.at[pl.ds(0, 8 * NB)], sem_w).wait()
            return c

        lax.fori_loop(0, KC, dk, jnp.int32(0))

    def do_range(lo_b, hi_b, table):
        count = hi_b - lo_b

        def gather(j, dst, sem):
            bb = (lo_b + j) * NB
            pltpu.async_copy(table.at[gl.at[pl.ds(bb, NB)]], dst, sem)

        def wait_gather(dst, sem):
            pltpu.make_async_copy(table.at[gl.at[pl.ds(0, NB)]], dst,
                                  sem).wait()

        @pl.when(count > 0)
        def _():
            gather(0, rows_a, sem_a)

        def body(p, carry):
            j0 = 2 * p
            j1 = j0 + 1

            @pl.when(j0 < count)
            def _():
                wait_gather(rows_a, sem_a)

            @pl.when(j1 < count)
            def _():
                gather(j1, rows_b, sem_b)

            @pl.when(j0 < count)
            def _():
                @pl.when(j0 >= 2)
                def _():
                    drain_writes(tiles_a, sem_wa)

                fill_tiles(lo_b + j0, rows_a, tiles_a, sem_wa)

            @pl.when(j0 + 2 < count)
            def _():
                gather(j0 + 2, rows_a, sem_a)

            @pl.when(j1 < count)
            def _():
                wait_gather(rows_b, sem_b)

                @pl.when(j1 >= 3)
                def _():
                    drain_writes(tiles_b, sem_wb)

                fill_tiles(lo_b + j1, rows_b, tiles_b, sem_wb)

            return carry

        lax.fori_loop(0, NBLK // 2 + 1, body, jnp.int32(0))

        @pl.when(count >= 1)
        def _():
            drain_writes(tiles_a, sem_wa)

        @pl.when(count >= 2)
        def _():
            drain_writes(tiles_b, sem_wb)

    split_lo = lax.shift_right_logical(m, 7)
    split_hi = lax.shift_right_logical(m + 127, 7)

    do_range(jnp.int32(0), split_lo, wold_hbm)
    do_range(split_hi, jnp.int32(NBLK), wnew_hbm)

    @pl.when(jnp.bitwise_and(m, 127) != 0)
    def _():
        sb = split_lo
        bb = sb * NB
        pltpu.async_copy(wold_hbm.at[gl.at[pl.ds(bb, NB)]], rows_a,
                         sem_a).wait()
        for g in range(8):
            v = gl[pl.ds(bb + 16 * g, 16)]
            pos = iota16 + (bb + 16 * g)
            is_new = pos >= (jnp.full((16,), 0, jnp.int32) + m)
            sidx[pl.ds(16 * g, 16)] = jnp.where(is_new, v, 0)
        pltpu.async_copy(wnew_hbm.at[sidx], rows_b, sem_b).wait()

        def tk(k, c):
            colb = 8 * k
            for cl in range(8):
                colv = jnp.full((16,), 0, jnp.int32) + (colb + cl)
                for g in range(8):
                    rowv = iota16 + (16 * g)
                    va = plsc.load_gather(rows_a, [rowv, colv])
                    vb = plsc.load_gather(rows_b, [rowv, colv])
                    keep_old = (rowv + bb) < (jnp.full((16,), 0, jnp.int32) + m)
                    va = jnp.where(keep_old, va, vb)
                    tiles_a[pl.ds(k * 1024 + cl * NB + 16 * g, 16)] = va
            tg = w * NBLK + sb
            pltpu.sync_copy(
                tiles_a.at[pl.ds(k * 1024, 8 * NB)],
                out_hbm.at[pl.ds((k * NT + tg) * (8 * NB), 8 * NB)])
            return c

        lax.fori_loop(0, KC, tk, jnp.int32(0))


def kernel(x, W_old, W_new):
    flat = x.reshape(-1).astype(jnp.int32)
    old_p, new_p, counts = _prep_kernel(flat)
    out1 = _main_kernel(W_old, W_new, old_p, new_p, counts)
    return (out1.reshape(KC, NT, 8, NB)
            .transpose(1, 3, 0, 2)
            .reshape(N, DIM))
